# baseline (speedup 1.0000x reference)
import functools

import jax
import jax.numpy as jnp
from jax import lax
from jax.experimental import pallas as pl
from jax.experimental.pallas import tpu as pltpu
from jax.experimental.pallas import tpu_sc as plsc

VOCAB = 1000
DIM = 64


@functools.lru_cache(maxsize=None)
def _make_sc_gather(B, T, D, V):
    info = plsc.get_sparse_core_info()
    NC, NS, L = info.num_cores, info.num_subcores, info.num_lanes
    NW = NC * NS
    BW = B // NW
    assert B % NW == 0 and BW % L == 0 and T % 2 == 0 and D % L == 0
    groups = BW // L
    mesh = plsc.VectorSubcoreMesh(core_axis_name="c", subcore_axis_name="s")

    @functools.partial(
        pl.kernel,
        mesh=mesh,
        compiler_params=pltpu.CompilerParams(needs_layout_passes=False),
        out_type=jax.ShapeDtypeStruct((T, D, B), jnp.float32),
        scratch_types=[
            pltpu.VMEM((V * D,), jnp.float32),
            pltpu.VMEM((BW * T,), jnp.int32),
            pltpu.VMEM((D, BW), jnp.float32),
            pltpu.VMEM((D, BW), jnp.float32),
            pltpu.SemaphoreType.DMA,
            pltpu.SemaphoreType.DMA,
        ],
    )
    def gather_kernel(x_hbm, tab_hbm, out_hbm, tabv, idxv, buf0, buf1,
                      sem0, sem1):
        wid = lax.axis_index("s") * NC + lax.axis_index("c")
        b0 = wid * BW
        pltpu.sync_copy(tab_hbm, tabv)
        pltpu.sync_copy(x_hbm.at[pl.ds(b0 * T, BW * T)], idxv)
        bufs = (buf0, buf1)
        sems = (sem0, sem1)
        lane = lax.iota(jnp.int32, L)
        rotc = [(jnp.full((L,), j, jnp.int32) - lane) % L for j in range(L)]
        mbit = [((lane >> s) & 1) == 1 for s in range(4)]

        def compute(t, buf):
            def one_group(g, buf):
                a_idx = (g * L + lane) * T + t
                rbase = plsc.load_gather(idxv, [a_idx]) * D
                for q in range(D // L):
                    base_q = rbase + q * L
                    s_regs = [plsc.load_gather(tabv, [base_q + rotc[j]])
                              for j in range(L)]
                    for s in range(4):
                        step = 1 << s
                        s_regs = [
                            jnp.where(mbit[s], s_regs[(j + step) % L],
                                      s_regs[j])
                            for j in range(L)
                        ]
                    for m in range(L):
                        buf[q * L + m, pl.ds(g * L, L)] = s_regs[m]

            def group_quad(gp, carry):
                for k in range(4):
                    one_group(4 * gp + k, buf)
                return carry

            lax.fori_loop(0, groups // 4, group_quad, 0)

        for p in (0, 1):
            compute(p, bufs[p])
            pltpu.async_copy(bufs[p], out_hbm.at[p, :, pl.ds(b0, BW)],
                             sems[p])

        def body(i, carry):
            for p in (0, 1):
                t = 2 * i + p
                pltpu.make_async_copy(
                    bufs[p], out_hbm.at[t - 2, :, pl.ds(b0, BW)], sems[p]
                ).wait()
                compute(t, bufs[p])
                pltpu.async_copy(bufs[p], out_hbm.at[t, :, pl.ds(b0, BW)],
                                 sems[p])
            return carry

        lax.fori_loop(1, T // 2, body, 0)

        for p in (0, 1):
            pltpu.make_async_copy(
                bufs[p], out_hbm.at[T - 2 + p, :, pl.ds(b0, BW)], sems[p]
            ).wait()

    return gather_kernel


def kernel(x, embedding):
    Bm, T = x.shape
    x1d = x.reshape(Bm * T).astype(jnp.int32)
    tab1d = embedding.reshape(VOCAB * DIM)
    out = _make_sc_gather(Bm, T, DIM, VOCAB)(x1d, tab1d)
    return jnp.transpose(out, (2, 0, 1))


# speedup vs baseline: 1.5734x; 1.5734x over previous
import functools

import jax
import jax.numpy as jnp
from jax import lax
from jax.experimental import pallas as pl
from jax.experimental.pallas import tpu as pltpu
from jax.experimental.pallas import tpu_sc as plsc

VOCAB = 1000
DIM = 64


@functools.lru_cache(maxsize=None)
def _make_sc_gather(B, T, D, V):
    info = plsc.get_sparse_core_info()
    NC, NS, L = info.num_cores, info.num_subcores, info.num_lanes
    NW = NC * NS
    BW = B // NW
    assert B % NW == 0 and BW % L == 0 and T % 2 == 0 and D % L == 0
    groups = BW // L
    mesh = plsc.VectorSubcoreMesh(core_axis_name="c", subcore_axis_name="s")

    @functools.partial(
        pl.kernel,
        mesh=mesh,
        compiler_params=pltpu.CompilerParams(needs_layout_passes=False),
        out_type=jax.ShapeDtypeStruct((T, D, B), jnp.float32),
        scratch_types=[
            pltpu.VMEM((V * D,), jnp.float32),
            pltpu.VMEM((BW * T,), jnp.int32),
            pltpu.VMEM((D, BW), jnp.float32),
            pltpu.VMEM((D, BW), jnp.float32),
            pltpu.SemaphoreType.DMA,
            pltpu.SemaphoreType.DMA,
        ],
    )
    def gather_kernel(x_hbm, tab_hbm, out_hbm, tabv, idxv, buf0, buf1,
                      sem0, sem1):
        wid = lax.axis_index("s") * NC + lax.axis_index("c")
        b0 = wid * BW
        pltpu.sync_copy(tab_hbm, tabv)
        pltpu.sync_copy(x_hbm.at[pl.ds(b0 * T, BW * T)], idxv)
        bufs = (buf0, buf1)
        sems = (sem0, sem1)
        lane = lax.iota(jnp.int32, L)
        rotc = [(jnp.full((L,), j, jnp.int32) - lane) % L for j in range(L)]
        mbit = [((lane >> s) & 1) == 1 for s in range(4)]

        def compute(t, buf):
            def one_group(g, buf):
                a_idx = (g * L + lane) * T + t
                rbase = plsc.load_gather(idxv, [a_idx]) * D
                for q in range(D // L):
                    base_q = rbase + q * L
                    s_regs = [plsc.load_gather(tabv, [base_q + rotc[j]])
                              for j in range(L)]
                    for s in range(4):
                        step = 1 << s
                        s_regs = [
                            jnp.where(mbit[s], s_regs[(j + step) % L],
                                      s_regs[j])
                            for j in range(L)
                        ]
                    for m in range(L):
                        buf[q * L + m, pl.ds(g * L, L)] = s_regs[m]

            def group_pair(gp, carry):
                one_group(2 * gp, buf)
                one_group(2 * gp + 1, buf)
                return carry

            lax.fori_loop(0, groups // 2, group_pair, 0)

        for p in (0, 1):
            compute(p, bufs[p])
            pltpu.async_copy(bufs[p], out_hbm.at[p, :, pl.ds(b0, BW)],
                             sems[p])

        def body(i, carry):
            for p in (0, 1):
                t = 2 * i + p
                pltpu.make_async_copy(
                    bufs[p], out_hbm.at[t - 2, :, pl.ds(b0, BW)], sems[p]
                ).wait()
                compute(t, bufs[p])
                pltpu.async_copy(bufs[p], out_hbm.at[t, :, pl.ds(b0, BW)],
                                 sems[p])
            return carry

        lax.fori_loop(1, T // 2, body, 0)

        for p in (0, 1):
            pltpu.make_async_copy(
                bufs[p], out_hbm.at[T - 2 + p, :, pl.ds(b0, BW)], sems[p]
            ).wait()

    return gather_kernel


def kernel(x, embedding):
    Bm, T = x.shape
    x1d = x.reshape(Bm * T).astype(jnp.int32)
    tab1d = embedding.reshape(VOCAB * DIM)
    out = _make_sc_gather(Bm, T, DIM, VOCAB)(x1d, tab1d)
    return jnp.transpose(out, (2, 0, 1))


# speedup vs baseline: 3.4316x; 2.1810x over previous
import functools

import jax
import jax.numpy as jnp
from jax import lax
from jax.experimental import pallas as pl
from jax.experimental.pallas import tpu as pltpu
from jax.experimental.pallas import tpu_sc as plsc

VOCAB = 1000
DIM = 64


@functools.lru_cache(maxsize=None)
def _make_sc_gather(B, T, D, V):
    info = plsc.get_sparse_core_info()
    NC, NS, L = info.num_cores, info.num_subcores, info.num_lanes
    NW = NC * NS
    BW = B // NW
    assert B % NW == 0 and BW % L == 0 and T % 2 == 0 and D % L == 0
    groups = BW // L
    mesh = plsc.VectorSubcoreMesh(core_axis_name="c", subcore_axis_name="s")

    @functools.partial(
        pl.kernel,
        mesh=mesh,
        compiler_params=pltpu.CompilerParams(needs_layout_passes=False),
        out_type=jax.ShapeDtypeStruct((T, D, B), jnp.float32),
        scratch_types=[
            pltpu.VMEM((V * D,), jnp.float32),
            pltpu.VMEM((BW * T,), jnp.int32),
            pltpu.VMEM((D, BW), jnp.float32),
            pltpu.VMEM((D, BW), jnp.float32),
            pltpu.SemaphoreType.DMA,
            pltpu.SemaphoreType.DMA,
        ],
    )
    def gather_kernel(x_hbm, tab_hbm, out_hbm, tabv, idxv, buf0, buf1,
                      sem0, sem1):
        wid = lax.axis_index("s") * NC + lax.axis_index("c")
        b0 = wid * BW
        pltpu.sync_copy(tab_hbm, tabv)
        pltpu.sync_copy(x_hbm.at[pl.ds(b0 * T, BW * T)], idxv)
        bufs = (buf0, buf1)
        sems = (sem0, sem1)
        lane = lax.iota(jnp.int32, L)
        rotc = [(jnp.full((L,), j, jnp.int32) - lane) % L for j in range(L)]
        mbit = [((lane >> s) & 1) == 1 for s in range(4)]

        def compute(t, buf):
            def one_group(g, buf):
                a_idx = (g * L + lane) * T + t
                rbase = plsc.load_gather(idxv, [a_idx]) * D
                for q in range(D // L):
                    base_q = rbase + q * L
                    s_regs = [plsc.load_gather(tabv, [base_q + rotc[j]])
                              for j in range(L)]
                    for s in range(4):
                        step = 1 << s
                        s_regs = [
                            jnp.where(mbit[s], s_regs[(j + step) % L],
                                      s_regs[j])
                            for j in range(L)
                        ]
                    for m in range(L):
                        buf[q * L + m, pl.ds(g * L, L)] = s_regs[m]

            @functools.partial(plsc.parallel_loop, 0, groups // 2)
            def group_pair(gp):
                one_group(2 * gp, buf)
                one_group(2 * gp + 1, buf)

        for p in (0, 1):
            compute(p, bufs[p])
            pltpu.async_copy(bufs[p], out_hbm.at[p, :, pl.ds(b0, BW)],
                             sems[p])

        def body(i, carry):
            for p in (0, 1):
                t = 2 * i + p
                pltpu.make_async_copy(
                    bufs[p], out_hbm.at[t - 2, :, pl.ds(b0, BW)], sems[p]
                ).wait()
                compute(t, bufs[p])
                pltpu.async_copy(bufs[p], out_hbm.at[t, :, pl.ds(b0, BW)],
                                 sems[p])
            return carry

        lax.fori_loop(1, T // 2, body, 0)

        for p in (0, 1):
            pltpu.make_async_copy(
                bufs[p], out_hbm.at[T - 2 + p, :, pl.ds(b0, BW)], sems[p]
            ).wait()

    return gather_kernel


def kernel(x, embedding):
    Bm, T = x.shape
    x1d = x.reshape(Bm * T).astype(jnp.int32)
    tab1d = embedding.reshape(VOCAB * DIM)
    out = _make_sc_gather(Bm, T, DIM, VOCAB)(x1d, tab1d)
    return jnp.transpose(out, (2, 0, 1))
